# Initial kernel scaffold; baseline (speedup 1.0000x reference)
import jax
import jax.numpy as jnp
from jax.experimental import pallas as pl


def kernel(x, gate_w, w_gate, w_up, w_down):
    raise NotImplementedError("write your pallas kernel here")


# speedup vs baseline: 1.8486x; 1.8486x over previous
import functools

import jax
import jax.numpy as jnp
from jax import lax
from jax.experimental import pallas as pl
from jax.experimental.pallas import tpu as pltpu
from jax.experimental.pallas import tpu_sc as plsc

E = 8
H = 2048
I = 1024
T = 4096
BM = 256
NB = 40
S = NB * BM
RB = 512
NRB = T // RB

NC, NS, L = 2, 16, 16
NW = NC * NS
TPW = T // NW
C = 16
NCH = TPW // C


def _router_body(x_ref, gw_ref, r0_ref, r1_ref, e0_ref, e1_ref,
                 w0_ref, w1_ref, cnt_ref, carry_ref):
    i = pl.program_id(0)

    @pl.when(i == 0)
    def _init():
        carry_ref[...] = jnp.zeros_like(carry_ref)

    @pl.when(i < NRB)
    def _work():
        xb = x_ref[...]
        gw = gw_ref[...]
        logits = lax.dot_general(xb, gw, (((1,), (1,)), ((), ())),
                                 preferred_element_type=jnp.float32)
        iota_e = lax.broadcasted_iota(jnp.int32, (RB, E), 1)
        m1 = jnp.max(logits, axis=1, keepdims=True)
        i1 = jnp.min(jnp.where(logits == m1, iota_e, E), axis=1)
        mask1 = iota_e == i1[:, None]
        l2 = jnp.where(mask1, -jnp.inf, logits)
        m2 = jnp.max(l2, axis=1, keepdims=True)
        i2 = jnp.min(jnp.where(l2 == m2, iota_e, E), axis=1)
        mask2 = iota_e == i2[:, None]
        w1 = jax.nn.sigmoid((m1 - m2)[:, 0])
        h1 = mask1.astype(jnp.float32)
        h2 = mask2.astype(jnp.float32)
        hsum = h1 + h2
        ri = lax.broadcasted_iota(jnp.int32, (RB, RB), 0)
        ci = lax.broadcasted_iota(jnp.int32, (RB, RB), 1)
        lt = (ci < ri).astype(jnp.float32)
        sexcl = lax.dot_general(lt, hsum, (((1,), (0,)), ((), ())),
                                preferred_element_type=jnp.float32)
        tot = sexcl + carry_ref[...].astype(jnp.float32)
        r0 = jnp.sum(tot * h1, axis=1)
        r1v = jnp.sum(tot * h2, axis=1)
        r0_ref[...] = r0.astype(jnp.int32).reshape(1, 1, RB)
        r1_ref[...] = r1v.astype(jnp.int32).reshape(1, 1, RB)
        e0_ref[...] = i1.reshape(1, 1, RB)
        e1_ref[...] = i2.reshape(1, 1, RB)
        w0_ref[...] = jnp.broadcast_to(w1[:, None], (RB, L)).reshape(1, RB, L)
        w1_ref[...] = jnp.broadcast_to((1.0 - w1)[:, None],
                                       (RB, L)).reshape(1, RB, L)
        carry_ref[...] = (carry_ref[...] +
                          jnp.sum(hsum, axis=0, keepdims=True).astype(jnp.int32))

    cnt_ref[...] = carry_ref[...]


def _router(xf, gate_w):
    blk = lambda i: (jnp.minimum(i, NRB - 1), 0, 0)
    return pl.pallas_call(
        _router_body,
        grid=(NRB + 1,),
        in_specs=[
            pl.BlockSpec((RB, H), lambda i: (jnp.minimum(i, NRB - 1), 0)),
            pl.BlockSpec((E, H), lambda i: (0, 0)),
        ],
        out_specs=[
            pl.BlockSpec((1, 1, RB), blk),
            pl.BlockSpec((1, 1, RB), blk),
            pl.BlockSpec((1, 1, RB), blk),
            pl.BlockSpec((1, 1, RB), blk),
            pl.BlockSpec((1, RB, L), blk),
            pl.BlockSpec((1, RB, L), blk),
            pl.BlockSpec((1, E), lambda i: (0, 0)),
        ],
        out_shape=[
            jax.ShapeDtypeStruct((NRB, 1, RB), jnp.int32),
            jax.ShapeDtypeStruct((NRB, 1, RB), jnp.int32),
            jax.ShapeDtypeStruct((NRB, 1, RB), jnp.int32),
            jax.ShapeDtypeStruct((NRB, 1, RB), jnp.int32),
            jax.ShapeDtypeStruct((NRB, RB, L), jnp.float32),
            jax.ShapeDtypeStruct((NRB, RB, L), jnp.float32),
            jax.ShapeDtypeStruct((1, E), jnp.int32),
        ],
        scratch_shapes=[pltpu.VMEM((1, E), jnp.int32)],
        compiler_params=pltpu.CompilerParams(
            dimension_semantics=("arbitrary",)),
    )(xf, gate_w)


def _pos_body(cnt_ref, e0_ref, r0_ref, e1_ref, r1_ref,
              p0_ref, p1_ref, bexp_ref):
    acc0 = r0_ref[...]
    acc1 = r1_ref[...]
    e0 = e0_ref[...]
    e1 = e1_ref[...]
    b_start = lax.broadcasted_iota(jnp.int32, (1, 64), 1) * BM
    bx = jnp.zeros((1, 64), jnp.int32)
    off = jnp.int32(0)
    for e in range(E):
        acc0 = acc0 + jnp.where(e0 == e, off, 0)
        acc1 = acc1 + jnp.where(e1 == e, off, 0)
        ce = cnt_ref[0, e]
        off = off + ((ce + BM - 1) // BM) * BM
        bx = bx + (b_start >= off).astype(jnp.int32)
    p0_ref[...] = acc0
    p1_ref[...] = acc1
    bexp_ref[...] = jnp.minimum(bx, E - 1)


def _pos(cnt, e0, r0, e1, r1):
    return pl.pallas_call(
        _pos_body,
        out_shape=[
            jax.ShapeDtypeStruct((32, 128), jnp.int32),
            jax.ShapeDtypeStruct((32, 128), jnp.int32),
            jax.ShapeDtypeStruct((1, 64), jnp.int32),
        ],
    )(cnt, e0, r0, e1, r1)


@functools.cache
def _sc_mesh():
    return plsc.VectorSubcoreMesh(core_axis_name="c", subcore_axis_name="s",
                                  num_cores=NC, num_subcores=NS)


def _dispatch_body(x_hbm, p0_hbm, p1_hbm, xs_hbm, idx0_v, idx1_v, buf, sem):
    wid = lax.axis_index("s") * NC + lax.axis_index("c")

    def body(j, _):
        base = wid * TPW + j * C
        pltpu.sync_copy(p0_hbm.at[pl.ds(base, C)], idx0_v)
        pltpu.sync_copy(p1_hbm.at[pl.ds(base, C)], idx1_v)
        pltpu.sync_copy(x_hbm.at[pl.ds(base, C)], buf)
        a = pltpu.async_copy(buf, xs_hbm.at[idx0_v], sem)
        b = pltpu.async_copy(buf, xs_hbm.at[idx1_v], sem)
        a.wait()
        b.wait()
        return 0

    lax.fori_loop(0, NCH, body, 0)


@functools.cache
def _dispatch():
    return pl.kernel(
        _dispatch_body,
        out_type=jax.ShapeDtypeStruct((S, H), jnp.float32),
        mesh=_sc_mesh(),
        scratch_types=[
            pltpu.VMEM((C,), jnp.int32),
            pltpu.VMEM((C,), jnp.int32),
            pltpu.VMEM((C, H), jnp.float32),
            pltpu.SemaphoreType.DMA,
        ],
    )


def _ffn_body(bexp_ref, xs_ref, wg_ref, wu_ref, wd_ref, out_ref):
    xb = xs_ref[...]
    g = lax.dot_general(xb, wg_ref[0], (((1,), (1,)), ((), ())),
                        preferred_element_type=jnp.float32)
    u = lax.dot_general(xb, wu_ref[0], (((1,), (1,)), ((), ())),
                        preferred_element_type=jnp.float32)
    h = g * jax.nn.sigmoid(g) * u
    out_ref[...] = lax.dot_general(h, wd_ref[0], (((1,), (1,)), ((), ())),
                                   preferred_element_type=jnp.float32)


def _grouped_ffn(bexp, xs, w_gate, w_up, w_down):
    grid_spec = pltpu.PrefetchScalarGridSpec(
        num_scalar_prefetch=1,
        grid=(NB,),
        in_specs=[
            pl.BlockSpec((BM, H), lambda b, be: (b, 0)),
            pl.BlockSpec((1, I, H), lambda b, be: (be[b], 0, 0)),
            pl.BlockSpec((1, I, H), lambda b, be: (be[b], 0, 0)),
            pl.BlockSpec((1, H, I), lambda b, be: (be[b], 0, 0)),
        ],
        out_specs=pl.BlockSpec((BM, H), lambda b, be: (b, 0)),
    )
    return pl.pallas_call(
        _ffn_body,
        grid_spec=grid_spec,
        out_shape=jax.ShapeDtypeStruct((S, H), jnp.float32),
        compiler_params=pltpu.CompilerParams(
            dimension_semantics=("arbitrary",)),
    )(bexp, xs, w_gate, w_up, w_down)


def _combine_body(hs_hbm, p0_hbm, p1_hbm, tw0_hbm, tw1_hbm, out_hbm,
                  idx0_v, idx1_v, w0_v, w1_v, bufa, bufb, bufo, sem):
    wid = lax.axis_index("s") * NC + lax.axis_index("c")

    def body(j, _):
        base = wid * TPW + j * C
        pltpu.sync_copy(p0_hbm.at[pl.ds(base, C)], idx0_v)
        pltpu.sync_copy(p1_hbm.at[pl.ds(base, C)], idx1_v)
        pltpu.sync_copy(tw0_hbm.at[pl.ds(base, C)], w0_v)
        pltpu.sync_copy(tw1_hbm.at[pl.ds(base, C)], w1_v)
        a = pltpu.async_copy(hs_hbm.at[idx0_v], bufa, sem)
        b = pltpu.async_copy(hs_hbm.at[idx1_v], bufb, sem)
        a.wait()
        b.wait()

        def row(rj, _):
            w0 = w0_v[rj, :]
            w1 = w1_v[rj, :]

            def col(cj, _):
                sl = pl.ds(cj * L, L)
                bufo[rj, sl] = w0 * bufa[rj, sl] + w1 * bufb[rj, sl]
                return 0

            lax.fori_loop(0, H // L, col, 0)
            return 0

        lax.fori_loop(0, C, row, 0)
        pltpu.sync_copy(bufo, out_hbm.at[pl.ds(base, C)])
        return 0

    lax.fori_loop(0, NCH, body, 0)


@functools.cache
def _combine():
    return pl.kernel(
        _combine_body,
        out_type=jax.ShapeDtypeStruct((T, H), jnp.float32),
        mesh=_sc_mesh(),
        scratch_types=[
            pltpu.VMEM((C,), jnp.int32),
            pltpu.VMEM((C,), jnp.int32),
            pltpu.VMEM((C, L), jnp.float32),
            pltpu.VMEM((C, L), jnp.float32),
            pltpu.VMEM((C, H), jnp.float32),
            pltpu.VMEM((C, H), jnp.float32),
            pltpu.VMEM((C, H), jnp.float32),
            pltpu.SemaphoreType.DMA,
        ],
    )


def kernel(x, gate_w, w_gate, w_up, w_down):
    bsz, seq, _ = x.shape
    xf = x.reshape(-1, H)
    r0, r1, e0, e1, tw0, tw1, cnt = _router(xf, gate_w)
    rs = lambda a: a.reshape(32, 128)
    p0, p1, bexp = _pos(cnt, rs(e0), rs(r0), rs(e1), rs(r1))
    p0f = p0.reshape(T)
    p1f = p1.reshape(T)
    xs = _dispatch()(xf, p0f, p1f)
    hs = _grouped_ffn(bexp[0, :NB], xs, w_gate, w_up, w_down)
    out = _combine()(hs, p0f, p1f, tw0.reshape(T, L), tw1.reshape(T, L))
    return out.reshape(bsz, seq, H)
